# baseline (speedup 1.0000x reference)
import jax
import jax.numpy as jnp
from jax import lax
from jax.experimental import pallas as pl
from jax.experimental.pallas import tpu as pltpu
from jax.experimental.pallas import tpu_sc as plsc

B = 16
N = 1000
L = 8000
LANES = 16
CHUNKS = L // LANES
NPAD = 1024
BOXPAD = 4096
GCH = 128
NGCH = NPAD // GCH


def _body(pred_hbm, ids_hbm, boxes_hbm, scores_hbm, classes_hbm, counts_hbm,
          sp_rows, ids_v, idxl_v, rows_v, boxes_v, scores_v, classes_v,
          counts_v, sem, sem2):
    s = lax.axis_index("s")
    b = s

    @pl.when(s == 0)
    def _():
        pltpu.sync_copy(pred_hbm, sp_rows)

    cp = pltpu.async_copy(ids_hbm, ids_v, sem)

    zi = jnp.zeros((LANES,), jnp.int32)
    for j in range(NGCH):
        for k in range(GCH // LANES):
            idxl_v[j, pl.ds(k * LANES, LANES)] = zi

    cp.wait()

    bf = b.astype(jnp.float32)
    _scan_scope = jax.named_scope("phase_scan")
    _scan_scope.__enter__()
    iota = lax.iota(jnp.int32, LANES)

    def step(i, off):
        base = i * LANES
        vb = ids_v[pl.ds(base, LANES)]
        mask = vb == bf
        incl = jnp.cumsum(jnp.where(mask, 1, 0).astype(jnp.int32))
        cnt = plsc.all_reduce_population_count(mask)
        ranks = off + incl - 1
        m2 = jnp.logical_and(mask, ranks < N)
        src = base + iota
        plsc.store_scatter(idxl_v, [ranks >> 7, ranks & 127], src, mask=m2)
        return off + cnt

    off = lax.fori_loop(0, CHUNKS, step, jnp.zeros((LANES,), jnp.int32))
    _scan_scope.__exit__(None, None, None)

    plsc.subcore_barrier()

    _g_scope = jax.named_scope("phase_gather")
    _g_scope.__enter__()
    gathers = []
    for j in range(NGCH):
        gathers.append(pltpu.async_copy(
            sp_rows.at[idxl_v.at[j]],
            rows_v.at[pl.ds(j * GCH, GCH)], sem2))
    for g in gathers:
        g.wait()
    _g_scope.__exit__(None, None, None)

    _e_scope = jax.named_scope("phase_emit")
    _e_scope.__enter__()
    cnt_eff = jnp.minimum(off, N)
    comp = iota & 3
    subslot = iota >> 2
    c5 = jnp.full((LANES,), 5, jnp.int32)
    c6 = jnp.full((LANES,), 6, jnp.int32)
    zf = jnp.zeros((LANES,), jnp.float32)

    def emit(j2, carry):
        slots = j2 * LANES + iota
        mv = slots < cnt_eff
        xs = plsc.load_gather(rows_v, [slots, c5])
        scores_v[pl.ds(j2 * LANES, LANES)] = jnp.where(mv, xs, zf)
        xc = plsc.load_gather(rows_v, [slots, c6])
        classes_v[pl.ds(j2 * LANES, LANES)] = jnp.where(
            mv, xc, zf).astype(jnp.int32)
        for t in range(4):
            bslot = j2 * LANES + t * 4 + subslot
            mb = bslot < cnt_eff
            xv = plsc.load_gather(rows_v, [bslot, 1 + comp])
            boxes_v[pl.ds(j2 * 64 + t * LANES, LANES)] = jnp.where(mb, xv, zf)
        return carry

    lax.fori_loop(0, NPAD // LANES, emit, 0)
    _e_scope.__exit__(None, None, None)

    counts_v[...] = off

    pltpu.sync_copy(boxes_v.at[pl.ds(0, 4 * N)], boxes_hbm.at[b])
    pltpu.sync_copy(scores_v.at[pl.ds(0, N)], scores_hbm.at[b])
    pltpu.sync_copy(classes_v.at[pl.ds(0, N)], classes_hbm.at[b])
    pltpu.sync_copy(counts_v.at[pl.ds(0, 8)], counts_hbm.at[b])


def kernel(predictions):
    ids = predictions[:, 0]
    pred8 = jnp.pad(predictions, ((0, 0), (0, 1)))
    mesh = plsc.VectorSubcoreMesh(
        core_axis_name="c", subcore_axis_name="s", num_cores=1)
    k = pl.kernel(
        _body,
        mesh=mesh,
        compiler_params=pltpu.CompilerParams(
            needs_layout_passes=False, use_tc_tiling_on_sc=False),
        out_type=[
            jax.ShapeDtypeStruct((B, 4 * N), jnp.float32),
            jax.ShapeDtypeStruct((B, N), jnp.float32),
            jax.ShapeDtypeStruct((B, N), jnp.int32),
            jax.ShapeDtypeStruct((B, 8), jnp.int32),
        ],
        scratch_types=[
            pltpu.VMEM_SHARED((L, 8), jnp.float32),
            pltpu.VMEM((L,), jnp.float32),
            pltpu.VMEM((NGCH, GCH), jnp.int32),
            pltpu.VMEM((NPAD, 8), jnp.float32),
            pltpu.VMEM((BOXPAD,), jnp.float32),
            pltpu.VMEM((NPAD,), jnp.float32),
            pltpu.VMEM((NPAD,), jnp.int32),
            pltpu.VMEM((LANES,), jnp.int32),
            pltpu.SemaphoreType.DMA,
            pltpu.SemaphoreType.DMA,
        ],
    )
    boxes, scores, classes, counts = k(pred8, ids)
    num_predictions = counts[:, :1]
    pred_boxes = boxes.reshape(B, N, 4)
    return (num_predictions, pred_boxes, scores, classes)


# speedup vs baseline: 1.0699x; 1.0699x over previous
import jax
import jax.numpy as jnp
from jax import lax
from jax.experimental import pallas as pl
from jax.experimental.pallas import tpu as pltpu
from jax.experimental.pallas import tpu_sc as plsc

B = 16
N = 1000
L = 8000
LANES = 16
CHUNKS = L // LANES
NPAD = 1024
BOXPAD = 4096


def _body(pt_hbm, boxes_hbm, scores_hbm, classes_hbm, counts_hbm,
          cols_v, boxes_v, scores_v, classes_v, counts_v, sem):
    c = lax.axis_index("c")
    s = lax.axis_index("s")

    @pl.when(s < 8)
    def _():
        b = c * 8 + s
        cp = pltpu.async_copy(pt_hbm, cols_v, sem)

        zf = jnp.zeros((LANES,), jnp.float32)
        zi = jnp.zeros((LANES,), jnp.int32)

        def zero_boxes(i, carry):
            boxes_v[pl.ds(i * LANES, LANES)] = zf
            return carry

        lax.fori_loop(0, BOXPAD // LANES, zero_boxes, 0)

        def zero_sc(i, carry):
            scores_v[pl.ds(i * LANES, LANES)] = zf
            classes_v[pl.ds(i * LANES, LANES)] = zi
            return carry

        lax.fori_loop(0, NPAD // LANES, zero_sc, 0)
        cp.wait()

        bf = b.astype(jnp.float32)

        def step(i, off):
            base = i * LANES
            vb = cols_v[pl.ds(base, LANES)]
            mask = vb == bf
            incl = jnp.cumsum(jnp.where(mask, 1, 0).astype(jnp.int32))
            cnt = jnp.max(incl)
            ranks = off + incl - 1
            m2 = jnp.logical_and(mask, ranks < N)
            idx4 = ranks * 4
            for col in range(4):
                x = cols_v[pl.ds((1 + col) * L + base, LANES)]
                plsc.store_scatter(boxes_v, [idx4 + col], x, mask=m2)
            xs = cols_v[pl.ds(5 * L + base, LANES)]
            plsc.store_scatter(scores_v, [ranks], xs, mask=m2)
            xc = cols_v[pl.ds(6 * L + base, LANES)].astype(jnp.int32)
            plsc.store_scatter(classes_v, [ranks], xc, mask=m2)
            return off + cnt

        off = lax.fori_loop(0, CHUNKS, step, jnp.zeros((LANES,), jnp.int32))
        for j in range(128 // LANES):
            counts_v[pl.ds(j * LANES, LANES)] = off

        pltpu.sync_copy(boxes_v, boxes_hbm.at[b])
        pltpu.sync_copy(scores_v, scores_hbm.at[b])
        pltpu.sync_copy(classes_v, classes_hbm.at[b])
        pltpu.sync_copy(counts_v, counts_hbm.at[b])


def kernel(predictions):
    pt = predictions.T.reshape(-1)
    mesh = plsc.VectorSubcoreMesh(core_axis_name="c", subcore_axis_name="s")
    k = pl.kernel(
        _body,
        mesh=mesh,
        compiler_params=pltpu.CompilerParams(needs_layout_passes=False),
        out_type=[
            jax.ShapeDtypeStruct((B, BOXPAD), jnp.float32),
            jax.ShapeDtypeStruct((B, NPAD), jnp.float32),
            jax.ShapeDtypeStruct((B, NPAD), jnp.int32),
            jax.ShapeDtypeStruct((B, 128), jnp.int32),
        ],
        scratch_types=[
            pltpu.VMEM((7 * L,), jnp.float32),
            pltpu.VMEM((BOXPAD,), jnp.float32),
            pltpu.VMEM((NPAD,), jnp.float32),
            pltpu.VMEM((NPAD,), jnp.int32),
            pltpu.VMEM((128,), jnp.int32),
            pltpu.SemaphoreType.DMA,
        ],
    )
    boxes, scores, classes, counts = k(pt)
    num_predictions = counts[:, :1]
    pred_boxes = boxes[:, :4 * N].reshape(B, N, 4)
    return (num_predictions, pred_boxes, scores[:, :N], classes[:, :N])


# speedup vs baseline: 1.0997x; 1.0279x over previous
import jax
import jax.numpy as jnp
from jax import lax
from jax.experimental import pallas as pl
from jax.experimental.pallas import tpu as pltpu
from jax.experimental.pallas import tpu_sc as plsc

B = 16
N = 1000
L = 8000
LANES = 16
CHUNKS = L // LANES
NPAD = 1024
BOXPAD = 4096


def _body(pt_hbm, boxes_hbm, scores_hbm, classes_hbm, counts_hbm,
          cols_v, boxes_v, scores_v, classes_v, counts_v, sem):
    b = lax.axis_index("s")
    if True:
        cp = pltpu.async_copy(pt_hbm, cols_v, sem)

        zf = jnp.zeros((LANES,), jnp.float32)
        zi = jnp.zeros((LANES,), jnp.int32)

        def zero_boxes(i, carry):
            boxes_v[pl.ds(i * LANES, LANES)] = zf
            return carry

        lax.fori_loop(0, BOXPAD // LANES, zero_boxes, 0)

        def zero_sc(i, carry):
            scores_v[pl.ds(i * LANES, LANES)] = zf
            classes_v[pl.ds(i * LANES, LANES)] = zi
            return carry

        lax.fori_loop(0, NPAD // LANES, zero_sc, 0)
        cp.wait()

        bf = b.astype(jnp.float32)

        def step(i, off):
            base = i * LANES
            vb = cols_v[pl.ds(base, LANES)]
            mask = vb == bf
            incl = jnp.cumsum(jnp.where(mask, 1, 0).astype(jnp.int32))
            cnt = jnp.max(incl)
            ranks = off + incl - 1
            m2 = jnp.logical_and(mask, ranks < N)
            idx4 = ranks * 4
            for col in range(4):
                x = cols_v[pl.ds((1 + col) * L + base, LANES)]
                plsc.store_scatter(boxes_v, [idx4 + col], x, mask=m2)
            xs = cols_v[pl.ds(5 * L + base, LANES)]
            plsc.store_scatter(scores_v, [ranks], xs, mask=m2)
            xc = cols_v[pl.ds(6 * L + base, LANES)].astype(jnp.int32)
            plsc.store_scatter(classes_v, [ranks], xc, mask=m2)
            return off + cnt

        off = lax.fori_loop(0, CHUNKS, step, jnp.zeros((LANES,), jnp.int32))
        for j in range(128 // LANES):
            counts_v[pl.ds(j * LANES, LANES)] = off

        pltpu.sync_copy(boxes_v, boxes_hbm.at[b])
        pltpu.sync_copy(scores_v, scores_hbm.at[b])
        pltpu.sync_copy(classes_v, classes_hbm.at[b])
        pltpu.sync_copy(counts_v, counts_hbm.at[b])


def kernel(predictions):
    pt = predictions.T.reshape(-1)
    mesh = plsc.VectorSubcoreMesh(
        core_axis_name="c", subcore_axis_name="s", num_cores=1)
    k = pl.kernel(
        _body,
        mesh=mesh,
        compiler_params=pltpu.CompilerParams(needs_layout_passes=False),
        out_type=[
            jax.ShapeDtypeStruct((B, BOXPAD), jnp.float32),
            jax.ShapeDtypeStruct((B, NPAD), jnp.float32),
            jax.ShapeDtypeStruct((B, NPAD), jnp.int32),
            jax.ShapeDtypeStruct((B, 128), jnp.int32),
        ],
        scratch_types=[
            pltpu.VMEM((7 * L,), jnp.float32),
            pltpu.VMEM((BOXPAD,), jnp.float32),
            pltpu.VMEM((NPAD,), jnp.float32),
            pltpu.VMEM((NPAD,), jnp.int32),
            pltpu.VMEM((128,), jnp.int32),
            pltpu.SemaphoreType.DMA,
        ],
    )
    boxes, scores, classes, counts = k(pt)
    num_predictions = counts[:, :1]
    pred_boxes = boxes[:, :4 * N].reshape(B, N, 4)
    return (num_predictions, pred_boxes, scores[:, :N], classes[:, :N])


# speedup vs baseline: 1.1872x; 1.0795x over previous
import jax
import jax.numpy as jnp
from jax import lax
from jax.experimental import pallas as pl
from jax.experimental.pallas import tpu as pltpu
from jax.experimental.pallas import tpu_sc as plsc

B = 16
N = 1000
L = 8000
LANES = 16
UNROLL = 4
CHUNKS = L // (LANES * UNROLL)
NPAD = 1024
BOXPAD = 4096


def _body(pt_hbm, boxes_hbm, scores_hbm, classes_hbm, counts_hbm,
          cols_v, boxes_v, scores_v, classes_v, counts_v, sem):
    b = lax.axis_index("s")
    if True:
        cp = pltpu.async_copy(pt_hbm, cols_v, sem)

        zf = jnp.zeros((LANES,), jnp.float32)
        zi = jnp.zeros((LANES,), jnp.int32)

        def zero_boxes(i, carry):
            boxes_v[pl.ds(i * LANES, LANES)] = zf
            return carry

        lax.fori_loop(0, BOXPAD // LANES, zero_boxes, 0)

        def zero_sc(i, carry):
            scores_v[pl.ds(i * LANES, LANES)] = zf
            classes_v[pl.ds(i * LANES, LANES)] = zi
            return carry

        lax.fori_loop(0, NPAD // LANES, zero_sc, 0)
        cp.wait()

        bf = b.astype(jnp.float32)

        def step(i, off):
            base0 = i * (LANES * UNROLL)
            masks, incls = [], []
            for k in range(UNROLL):
                vb = cols_v[pl.ds(base0 + k * LANES, LANES)]
                masks.append(vb == bf)
            for k in range(UNROLL):
                incls.append(
                    jnp.cumsum(jnp.where(masks[k], 1, 0).astype(jnp.int32)))
            for k in range(UNROLL):
                base = base0 + k * LANES
                ranks = off + incls[k] - 1
                m2 = jnp.logical_and(masks[k], ranks < N)
                idx4 = ranks * 4
                for col in range(4):
                    x = cols_v[pl.ds((1 + col) * L + base, LANES)]
                    plsc.store_scatter(boxes_v, [idx4 + col], x, mask=m2)
                xs = cols_v[pl.ds(5 * L + base, LANES)]
                plsc.store_scatter(scores_v, [ranks], xs, mask=m2)
                xc = cols_v[pl.ds(6 * L + base, LANES)].astype(jnp.int32)
                plsc.store_scatter(classes_v, [ranks], xc, mask=m2)
                off = off + jnp.max(incls[k])
            return off

        off = lax.fori_loop(0, CHUNKS, step, jnp.zeros((LANES,), jnp.int32))
        for j in range(128 // LANES):
            counts_v[pl.ds(j * LANES, LANES)] = off

        pltpu.sync_copy(boxes_v, boxes_hbm.at[b])
        pltpu.sync_copy(scores_v, scores_hbm.at[b])
        pltpu.sync_copy(classes_v, classes_hbm.at[b])
        pltpu.sync_copy(counts_v, counts_hbm.at[b])


def kernel(predictions):
    pt = predictions.T.reshape(-1)
    mesh = plsc.VectorSubcoreMesh(
        core_axis_name="c", subcore_axis_name="s", num_cores=1)
    k = pl.kernel(
        _body,
        mesh=mesh,
        compiler_params=pltpu.CompilerParams(needs_layout_passes=False),
        out_type=[
            jax.ShapeDtypeStruct((B, BOXPAD), jnp.float32),
            jax.ShapeDtypeStruct((B, NPAD), jnp.float32),
            jax.ShapeDtypeStruct((B, NPAD), jnp.int32),
            jax.ShapeDtypeStruct((B, 128), jnp.int32),
        ],
        scratch_types=[
            pltpu.VMEM((7 * L,), jnp.float32),
            pltpu.VMEM((BOXPAD,), jnp.float32),
            pltpu.VMEM((NPAD,), jnp.float32),
            pltpu.VMEM((NPAD,), jnp.int32),
            pltpu.VMEM((128,), jnp.int32),
            pltpu.SemaphoreType.DMA,
        ],
    )
    boxes, scores, classes, counts = k(pt)
    num_predictions = counts[:, :1]
    pred_boxes = boxes[:, :4 * N].reshape(B, N, 4)
    return (num_predictions, pred_boxes, scores[:, :N], classes[:, :N])
